# baseline (speedup 1.0000x reference)
import functools

import jax
import jax.numpy as jnp
from jax import lax
from jax.experimental import pallas as pl
from jax.experimental.pallas import tpu as pltpu
from jax.experimental.pallas import tpu_sc as plsc

K = 16
BR = 2048


def _knn_body(xr, xc, dist_ref, edge_ref, b, j, n, k, br, same):
    sq_r = jnp.sum(xr * xr, axis=1, keepdims=True)
    sq_c = sq_r if same else jnp.sum(xc * xc, axis=1, keepdims=True)
    inner2 = jax.lax.dot_general(
        xr, xc + xc, (((1,), (1,)), ((), ())),
        preferred_element_type=jnp.float32)
    e = sq_c.T - inner2
    inf = jnp.float32(jnp.inf)
    iota = jax.lax.broadcasted_iota(jnp.int32, e.shape, 1)
    rowid = jax.lax.broadcasted_iota(jnp.int32, e.shape, 0) + j * br
    m1 = jnp.min(jnp.where(iota == rowid, inf, e), axis=1, keepdims=True)
    q = m1 - 1.0
    kb = jax.lax.bitcast_convert_type(e - q, jnp.int32)
    keys = jax.lax.bitcast_convert_type((kb & jnp.int32(-2048)) | iota,
                                        jnp.float32)
    m = jnp.min(keys, axis=1, keepdims=True)
    ms = [m]
    for _ in range(k - 1):
        m = jnp.min(jnp.where(keys > m, keys, inf), axis=1, keepdims=True)
        ms.append(m)
    packed = jnp.concatenate(ms, axis=1)
    pi = jax.lax.bitcast_convert_type(packed, jnp.int32)
    vals = jax.lax.bitcast_convert_type(pi & jnp.int32(-2048), jnp.float32)
    dist_ref[0] = vals + (sq_r + q)
    src = (pi & jnp.int32(2047)) + b * n
    row0 = b * n + j * br
    dst = (jax.lax.broadcasted_iota(jnp.int32, (br, k), 0) + row0)
    edge_ref[0, 0] = src
    edge_ref[1, 0] = dst


def _knn_kernel2(xr_ref, xc_ref, dist_ref, edge_ref, *, n, k, br):
    _knn_body(xr_ref[0], xc_ref[0], dist_ref, edge_ref,
              pl.program_id(0), pl.program_id(1), n, k, br, same=False)


def _knn_kernel1(xc_ref, dist_ref, edge_ref, *, n, k):
    x = xc_ref[0]
    _knn_body(x, x, dist_ref, edge_ref,
              pl.program_id(0), 0, n, k, n, same=True)


def _knn_topk(x):
    b, n, c = x.shape
    br = min(BR, n)
    out_shape = [
        jax.ShapeDtypeStruct((b, n, K), jnp.float32),
        jax.ShapeDtypeStruct((2, b, n, K), jnp.int32),
    ]
    if br == n:
        return pl.pallas_call(
            functools.partial(_knn_kernel1, n=n, k=K),
            grid=(b,),
            in_specs=[pl.BlockSpec((1, n, c), lambda i: (i, 0, 0))],
            out_specs=[
                pl.BlockSpec((1, n, K), lambda i: (i, 0, 0)),
                pl.BlockSpec((2, 1, n, K), lambda i: (0, i, 0, 0)),
            ],
            out_shape=out_shape,
            compiler_params=pltpu.CompilerParams(
                dimension_semantics=("parallel",)),
        )(x)
    return pl.pallas_call(
        functools.partial(_knn_kernel2, n=n, k=K, br=br),
        grid=(b, n // br),
        in_specs=[
            pl.BlockSpec((1, br, c), lambda i, j: (i, j, 0)),
            pl.BlockSpec((1, n, c), lambda i, j: (i, 0, 0)),
        ],
        out_specs=[
            pl.BlockSpec((1, br, K), lambda i, j: (i, j, 0)),
            pl.BlockSpec((2, 1, br, K), lambda i, j: (0, i, j, 0)),
        ],
        out_shape=out_shape,
        compiler_params=pltpu.CompilerParams(
            dimension_semantics=("parallel", "parallel")),
    )(x, x)


def _dst_sc(total_edges):
    info = plsc.get_sparse_core_info()
    nw = info.num_cores * info.num_subcores
    per_w = total_edges // nw
    vecs = per_w // 16

    @functools.partial(
        pl.kernel,
        mesh=plsc.VectorSubcoreMesh(core_axis_name="c", subcore_axis_name="s"),
        out_type=jax.ShapeDtypeStruct((total_edges,), jnp.int32),
        scratch_types=[pltpu.VMEM((per_w,), jnp.int32)],
    )
    def dst_kernel(out_hbm, buf):
        wid = lax.axis_index("s") * info.num_cores + lax.axis_index("c")
        row0 = wid * vecs

        def body(i, carry):
            buf[pl.ds(i * 16, 16)] = jnp.full((16,), row0 + i, jnp.int32)
            return carry

        lax.fori_loop(0, vecs, body, 0)
        pltpu.sync_copy(buf, out_hbm.at[pl.ds(wid * per_w, per_w)])

    return dst_kernel()


def kernel(x):
    b, n, c = x.shape
    knn_dists, edge = _knn_topk(x)
    edge_index = edge.reshape(2, b * n * K)
    dst = _dst_sc(b * n * K)
    edge_index = edge_index.at[1].set(dst)
    return edge_index, knn_dists, b * n


# speedup vs baseline: 1.0008x; 1.0008x over previous
import functools

import jax
import jax.numpy as jnp
from jax import lax
from jax.experimental import pallas as pl
from jax.experimental.pallas import tpu as pltpu
from jax.experimental.pallas import tpu_sc as plsc

K = 16
BR = 2048


def _knn_body(xr, xc, dist_ref, edge_ref, b, j, n, k, br, same):
    sq_r = jnp.sum(xr * xr, axis=1, keepdims=True)
    sq_c = sq_r if same else jnp.sum(xc * xc, axis=1, keepdims=True)
    inner2 = jax.lax.dot_general(
        xr, xc + xc, (((1,), (1,)), ((), ())),
        preferred_element_type=jnp.float32)
    e = sq_c.T - inner2
    inf = jnp.float32(jnp.inf)
    iota = jax.lax.broadcasted_iota(jnp.int32, e.shape, 1)
    rowid = jax.lax.broadcasted_iota(jnp.int32, e.shape, 0) + j * br
    m1 = jnp.min(jnp.where(iota == rowid, inf, e), axis=1, keepdims=True)
    q = m1 - 1.0
    kb = jax.lax.bitcast_convert_type(e - q, jnp.int32)
    keys = jax.lax.bitcast_convert_type((kb & jnp.int32(-2048)) | iota,
                                        jnp.float32)
    m = jnp.min(keys, axis=1, keepdims=True)
    ms = [m]
    for _ in range(k - 1):
        m = jnp.min(jnp.where(keys > m, keys, inf), axis=1, keepdims=True)
        ms.append(m)
    packed = jnp.concatenate(ms, axis=1)
    pi = jax.lax.bitcast_convert_type(packed, jnp.int32)
    vals = jax.lax.bitcast_convert_type(pi & jnp.int32(-2048), jnp.float32)
    dist_ref[0] = vals + (sq_r + q)
    edge_ref[0] = (pi & jnp.int32(2047)) + b * n


def _knn_kernel2(xr_ref, xc_ref, dist_ref, edge_ref, *, n, k, br):
    _knn_body(xr_ref[0], xc_ref[0], dist_ref, edge_ref,
              pl.program_id(0), pl.program_id(1), n, k, br, same=False)


def _knn_kernel1(xc_ref, dist_ref, edge_ref, *, n, k):
    x = xc_ref[0]
    _knn_body(x, x, dist_ref, edge_ref,
              pl.program_id(0), 0, n, k, n, same=True)


def _knn_topk(x):
    b, n, c = x.shape
    br = min(BR, n)
    out_shape = [
        jax.ShapeDtypeStruct((b, n, K), jnp.float32),
        jax.ShapeDtypeStruct((b, n, K), jnp.int32),
    ]
    if br == n:
        return pl.pallas_call(
            functools.partial(_knn_kernel1, n=n, k=K),
            grid=(b,),
            in_specs=[pl.BlockSpec((1, n, c), lambda i: (i, 0, 0))],
            out_specs=[
                pl.BlockSpec((1, n, K), lambda i: (i, 0, 0)),
                pl.BlockSpec((1, n, K), lambda i: (i, 0, 0)),
            ],
            out_shape=out_shape,
            compiler_params=pltpu.CompilerParams(
                dimension_semantics=("parallel",)),
        )(x)
    return pl.pallas_call(
        functools.partial(_knn_kernel2, n=n, k=K, br=br),
        grid=(b, n // br),
        in_specs=[
            pl.BlockSpec((1, br, c), lambda i, j: (i, j, 0)),
            pl.BlockSpec((1, n, c), lambda i, j: (i, 0, 0)),
        ],
        out_specs=[
            pl.BlockSpec((1, br, K), lambda i, j: (i, j, 0)),
            pl.BlockSpec((1, br, K), lambda i, j: (i, j, 0)),
        ],
        out_shape=out_shape,
        compiler_params=pltpu.CompilerParams(
            dimension_semantics=("parallel", "parallel")),
    )(x, x)


def _edge_sc(src_flat):
    total = src_flat.shape[0]
    info = plsc.get_sparse_core_info()
    nw = info.num_cores * info.num_subcores
    per_w = total // nw
    vecs = per_w // 16
    src2 = src_flat.reshape(nw, per_w)

    @functools.partial(
        pl.kernel,
        mesh=plsc.VectorSubcoreMesh(core_axis_name="c", subcore_axis_name="s"),
        out_type=jax.ShapeDtypeStruct((2, nw, per_w), jnp.int32),
        scratch_types=[pltpu.VMEM((per_w,), jnp.int32)],
    )
    def edge_kernel(src_hbm, out_hbm, buf):
        wid = lax.axis_index("s") * info.num_cores + lax.axis_index("c")
        pltpu.sync_copy(src_hbm.at[wid], out_hbm.at[0, wid])
        row0 = wid * vecs

        def body(i, carry):
            buf[pl.ds(i * 16, 16)] = jnp.full((16,), row0 + i, jnp.int32)
            return carry

        lax.fori_loop(0, vecs, body, 0)
        pltpu.sync_copy(buf, out_hbm.at[1, wid])

    return edge_kernel(src2).reshape(2, total)


def kernel(x):
    b, n, c = x.shape
    knn_dists, src = _knn_topk(x)
    edge_index = _edge_sc(src.reshape(-1))
    return edge_index, knn_dists, b * n
